# baseline (speedup 1.0000x reference)
import functools

import jax
import jax.numpy as jnp
from jax import lax
from jax.experimental import pallas as pl
from jax.experimental.pallas import tpu as pltpu
from jax.experimental.pallas import tpu_sc as plsc

NC = 2
NS = 16
NW = NC * NS
G = 128
NBUF = 8


def _mesh():
    return plsc.VectorSubcoreMesh(
        core_axis_name="c", subcore_axis_name="s", num_cores=NC, num_subcores=NS
    )


_SC_PARAMS = pltpu.CompilerParams(use_tc_tiling_on_sc=False)


def _core_split(ci, si, GP0, GP1):
    gpc = jnp.where(ci == 0, GP0, GP1)
    base = jnp.where(ci == 0, si * GP0, NS * GP0 + si * GP1)
    return gpc, base


def _make_deg(H, NP, GP0, GP1):
    RPS = NP // NS
    GPM = max(GP0, GP1)

    @functools.partial(
        pl.kernel,
        mesh=_mesh(),
        compiler_params=_SC_PARAMS,
        out_type=jax.ShapeDtypeStruct((NC, NP, H), jnp.float32),
        scratch_types=[
            pltpu.VMEM((GPM, G), jnp.int32),
            pltpu.VMEM((G, H), jnp.float32),
            pltpu.VMEM((RPS, H), jnp.float32),
            pltpu.VMEM_SHARED((NP, H), jnp.float32),
            pltpu.SemaphoreType.DMA((NBUF,)),
        ],
    )
    def k(dst_hbm, out_hbm, dstb, ones, zbuf, acc, ssem):
        ci = lax.axis_index("c")
        si = lax.axis_index("s")
        gpc, base = _core_split(ci, si, GP0, GP1)

        def zfill(i, _):
            zbuf[i, :] = jnp.zeros((H,), jnp.float32)
            return 0

        lax.fori_loop(0, RPS, zfill, 0)

        def ofill(i, _):
            ones[i, :] = jnp.ones((H,), jnp.float32)
            return 0

        lax.fori_loop(0, G, ofill, 0)

        @pl.when(ci == 0)
        def _():
            pltpu.sync_copy(dst_hbm.at[pl.ds(base, GP0)],
                            dstb.at[pl.ds(0, GP0)])

        @pl.when(ci == 1)
        def _():
            pltpu.sync_copy(dst_hbm.at[pl.ds(base, GP1)],
                            dstb.at[pl.ds(0, GP1)])

        pltpu.sync_copy(zbuf, acc.at[pl.ds(si * RPS, RPS)])
        plsc.subcore_barrier()

        for b in range(NBUF):
            pltpu.async_copy(ones, acc.at[dstb.at[b]], ssem.at[b], add=True)

        def outer(t, _):
            t0 = t * NBUF
            for b in range(NBUF):
                g = t0 + b
                pltpu.make_async_copy(ones, acc.at[dstb.at[b]],
                                      ssem.at[b]).wait()
                pltpu.async_copy(ones, acc.at[dstb.at[g]], ssem.at[b],
                                 add=True)
            return 0

        lax.fori_loop(1, gpc // NBUF, outer, 0)
        for b in range(NBUF):
            pltpu.make_async_copy(ones, acc.at[dstb.at[b]], ssem.at[b]).wait()
        plsc.subcore_barrier()
        pltpu.sync_copy(acc.at[pl.ds(si * RPS, RPS)],
                        out_hbm.at[ci, pl.ds(si * RPS, RPS)])

    return k


def _make_rowsum(H, NP, GP0, GP1):
    RPS = NP // NS
    GPM = max(GP0, GP1)

    @functools.partial(
        pl.kernel,
        mesh=_mesh(),
        compiler_params=_SC_PARAMS,
        out_type=jax.ShapeDtypeStruct((NC, NP, H), jnp.float32),
        scratch_types=[
            pltpu.VMEM((GPM, G), jnp.int32),
            pltpu.VMEM((GPM, G), jnp.int32),
            pltpu.VMEM((NBUF, G, H), jnp.float32),
            pltpu.VMEM((RPS, H), jnp.float32),
            pltpu.VMEM_SHARED((NP, H), jnp.float32),
            pltpu.SemaphoreType.DMA((NBUF,)),
            pltpu.SemaphoreType.DMA((NBUF,)),
        ],
    )
    def k(y_hbm, src_hbm, dst_hbm, out_hbm, srcb, dstb, rows, zbuf, acc,
          gsem, ssem):
        ci = lax.axis_index("c")
        si = lax.axis_index("s")
        gpc, base = _core_split(ci, si, GP0, GP1)

        def zfill(i, _):
            zbuf[i, :] = jnp.zeros((H,), jnp.float32)
            return 0

        lax.fori_loop(0, RPS, zfill, 0)

        def gather_work():
            for b in range(NBUF):
                pltpu.async_copy(y_hbm.at[srcb.at[b]], rows.at[b], gsem.at[b])
            pltpu.sync_copy(zbuf, acc.at[pl.ds(si * RPS, RPS)])
            plsc.subcore_barrier()

            def outer(t, _):
                t0 = t * NBUF
                for b in range(NBUF):
                    g = t0 + b
                    pltpu.make_async_copy(
                        y_hbm.at[srcb.at[g]], rows.at[b], gsem.at[b]).wait()
                    pltpu.async_copy(rows.at[b], acc.at[dstb.at[g]],
                                     ssem.at[b], add=True)
                for b in range(NBUF):
                    ng = t0 + NBUF + b

                    @pl.when(ng < gpc)
                    def _():
                        pltpu.make_async_copy(
                            rows.at[b], acc.at[dstb.at[b]], ssem.at[b]).wait()
                        pltpu.async_copy(y_hbm.at[srcb.at[ng]], rows.at[b],
                                         gsem.at[b])

                return 0

            lax.fori_loop(0, gpc // NBUF, outer, 0)
            for b in range(NBUF):
                pltpu.make_async_copy(rows.at[b], acc.at[dstb.at[b]],
                                      ssem.at[b]).wait()

        if GP1 == 0:
            @pl.when(ci == 0)
            def _():
                pltpu.sync_copy(src_hbm.at[pl.ds(base, GP0)],
                                srcb.at[pl.ds(0, GP0)])
                pltpu.sync_copy(dst_hbm.at[pl.ds(base, GP0)],
                                dstb.at[pl.ds(0, GP0)])
                gather_work()

            @pl.when(ci == 1)
            def _():
                pltpu.sync_copy(zbuf, acc.at[pl.ds(si * RPS, RPS)])
                plsc.subcore_barrier()
        else:
            @pl.when(ci == 0)
            def _():
                pltpu.sync_copy(src_hbm.at[pl.ds(base, GP0)],
                                srcb.at[pl.ds(0, GP0)])
                pltpu.sync_copy(dst_hbm.at[pl.ds(base, GP0)],
                                dstb.at[pl.ds(0, GP0)])

            @pl.when(ci == 1)
            def _():
                pltpu.sync_copy(src_hbm.at[pl.ds(base, GP1)],
                                srcb.at[pl.ds(0, GP1)])
                pltpu.sync_copy(dst_hbm.at[pl.ds(base, GP1)],
                                dstb.at[pl.ds(0, GP1)])

            gather_work()
        plsc.subcore_barrier()
        pltpu.sync_copy(acc.at[pl.ds(si * RPS, RPS)],
                        out_hbm.at[ci, pl.ds(si * RPS, RPS)])

    return k


def _k_xw(x, W1, N, NP, R):
    D = x.shape[1]
    H = W1.shape[1]
    grid = N // R

    def body(x_ref, w_ref, xw_ref):
        xw_ref[...] = jnp.dot(x_ref[...], w_ref[...],
                              preferred_element_type=jnp.float32)

    return pl.pallas_call(
        body,
        grid=(grid,),
        in_specs=[
            pl.BlockSpec((R, D), lambda i: (i, 0)),
            pl.BlockSpec((D, H), lambda i: (0, 0)),
        ],
        out_specs=pl.BlockSpec((R, H), lambda i: (i, 0)),
        out_shape=jax.ShapeDtypeStruct((NP, H), jnp.float32),
    )(x, W1)


def _k_scale(degbp, xwp, RP):
    NR = xwp.shape[0]
    grid = NR // RP

    def body(deg_ref, xw_ref, dinv_ref, y_ref):
        deg = deg_ref[0] + deg_ref[1] + 1.0
        dinvb = lax.rsqrt(deg)
        dinv_ref[...] = dinvb
        y_ref[...] = xw_ref[...] * dinvb

    return pl.pallas_call(
        body,
        grid=(grid,),
        in_specs=[
            pl.BlockSpec((NC, RP, 128), lambda i: (0, i, 0)),
            pl.BlockSpec((RP, 128), lambda i: (i, 0)),
        ],
        out_specs=[
            pl.BlockSpec((RP, 128), lambda i: (i, 0)),
            pl.BlockSpec((RP, 128), lambda i: (i, 0)),
        ],
        out_shape=[
            jax.ShapeDtypeStruct((NR, 128), jnp.float32),
            jax.ShapeDtypeStruct((NR, 128), jnp.float32),
        ],
    )(degbp, xwp)


def _k_layer(Sp, yp, dinvbp, bp, Wp, RP, with_hs, N):
    NR = yp.shape[0]
    grid = NR // RP

    def body(s_ref, y_ref, dinv_ref, b_ref, w_ref, o_ref, *hs_ref):
        S = s_ref[0] + s_ref[1]
        dinvb = dinv_ref[...]
        h = jnp.maximum(dinvb * (S + y_ref[...]) + b_ref[...], 0.0)
        o_ref[...] = jnp.dot(h, w_ref[...],
                             preferred_element_type=jnp.float32) * dinvb
        if with_hs:
            i = pl.program_id(0)
            r = lax.broadcasted_iota(jnp.int32, (RP, 128), 0) + i * RP
            nid = r * 8 + lax.broadcasted_iota(jnp.int32, (RP, 128), 1) // 16
            hm = jnp.where(nid < N, h, 0.0)
            hs_ref[0][...] = jnp.sum(hm, axis=0, keepdims=True)[None]

    out_specs = [pl.BlockSpec((RP, 128), lambda i: (i, 0))]
    out_shape = [jax.ShapeDtypeStruct((NR, 128), jnp.float32)]
    if with_hs:
        out_specs.append(pl.BlockSpec((1, 1, 128), lambda i: (i, 0, 0)))
        out_shape.append(jax.ShapeDtypeStruct((grid, 1, 128), jnp.float32))

    res = pl.pallas_call(
        body,
        grid=(grid,),
        in_specs=[
            pl.BlockSpec((NC, RP, 128), lambda i: (0, i, 0)),
            pl.BlockSpec((RP, 128), lambda i: (i, 0)),
            pl.BlockSpec((RP, 128), lambda i: (i, 0)),
            pl.BlockSpec((1, 128), lambda i: (0, 0)),
            pl.BlockSpec((128, 128), lambda i: (0, 0)),
        ],
        out_specs=out_specs,
        out_shape=out_shape,
    )(Sp, yp, dinvbp, bp, Wp)
    return res if with_hs else (res[0], None)


def _stage4(S3p, y3bp, dinvbp, b3r, hs, Wvp, bvr, N):
    NR = y3bp.shape[0]
    KB = hs.shape[0]

    def body(s_ref, y_ref, dinv_ref, b_ref, hs_ref, wv_ref, bv_ref,
             choice_ref, value_ref):
        S = s_ref[0] + s_ref[1]
        c = dinv_ref[...] * (S + y_ref[...]) + b_ref[0, 0]
        r = lax.broadcasted_iota(jnp.int32, (NR, 128), 0)
        nid = r * 8 + lax.broadcasted_iota(jnp.int32, (NR, 128), 1) // 16
        valid = nid < N
        c = jnp.where(valid, c, -jnp.inf)
        m = jnp.max(c)
        e = jnp.where(valid, jnp.exp(c - m), 0.0)
        choice_ref[...] = e * (16.0 / jnp.sum(e))
        hsum = jnp.sum(hs_ref[...], axis=0)
        value_ref[...] = (
            jnp.dot(hsum, wv_ref[...], preferred_element_type=jnp.float32)
            * (1.0 / N) + bv_ref[...]
        )

    return pl.pallas_call(
        body,
        in_specs=[
            pl.BlockSpec((NC, NR, 128), lambda: (0, 0, 0)),
            pl.BlockSpec((NR, 128), lambda: (0, 0)),
            pl.BlockSpec((NR, 128), lambda: (0, 0)),
            pl.BlockSpec((1, 1), lambda: (0, 0)),
            pl.BlockSpec((KB, 1, 128), lambda: (0, 0, 0)),
            pl.BlockSpec((128, 1), lambda: (0, 0)),
            pl.BlockSpec((1, 1), lambda: (0, 0)),
        ],
        out_specs=[
            pl.BlockSpec((NR, 128), lambda: (0, 0)),
            pl.BlockSpec((1, 1), lambda: (0, 0)),
        ],
        out_shape=[
            jax.ShapeDtypeStruct((NR, 128), jnp.float32),
            jax.ShapeDtypeStruct((1, 1), jnp.float32),
        ],
    )(S3p, y3bp, dinvbp, b3r, hs, Wvp, bvr)


def kernel(x, edge_index, W1, b1, W2, b2, W3, b3, Wv, bv):
    N, D = x.shape
    E = edge_index.shape[1]
    H = W1.shape[1]

    NP = ((N + 1 + 2047) // 2048) * 2048
    NR = NP * H // 128
    RP = NR // 5
    R = 2000
    TGS = ((-(-E // (NW * G)) + 7) // 8) * 8
    GP0 = 2 * TGS
    GP1 = 0
    DGP0 = ((TGS * 6 // 5) // 8) * 8
    DGP1 = 2 * TGS - DGP0
    EP = NS * 2 * TGS * G

    src = edge_index[0]
    dst = edge_index[1]
    pad = EP - E
    srcp = jnp.concatenate([src, jnp.zeros((pad,), jnp.int32)]).reshape(-1, G)
    dstp = jnp.concatenate([dst, jnp.full((pad,), N, jnp.int32)]).reshape(-1, G)

    eye8 = jnp.eye(8, dtype=jnp.float32)
    W2p = jnp.kron(eye8, W2)
    W3bp = jnp.kron(eye8, jnp.tile(W3, (1, H)))
    b1p = jnp.tile(b1, 8).reshape(1, 128)
    b2p = jnp.tile(b2, 8).reshape(1, 128)
    Wvp = jnp.tile(Wv, (8, 1))

    degb = _make_deg(H, NP, DGP0, DGP1)(dstp)
    xw = _k_xw(x, W1, N, NP, R)
    dinvbp, y1p = _k_scale(degb.reshape(NC, NR, 128), xw.reshape(NR, 128), RP)
    rowsum = _make_rowsum(H, NP, GP0, GP1)
    S1p = rowsum(y1p.reshape(NP, H), srcp, dstp)
    y2p, _ = _k_layer(S1p.reshape(NC, NR, 128), y1p, dinvbp, b1p, W2p,
                      RP, False, N)
    S2p = rowsum(y2p.reshape(NP, H), srcp, dstp)
    y3bp, hs = _k_layer(S2p.reshape(NC, NR, 128), y2p, dinvbp, b2p, W3bp,
                        RP, True, N)
    S3p = rowsum(y3bp.reshape(NP, H), srcp, dstp)
    choicebp, value = _stage4(
        S3p.reshape(NC, NR, 128), y3bp, dinvbp, b3.reshape(1, 1), hs,
        Wvp, bv.reshape(1, 1), N,
    )
    return choicebp.reshape(NP, H)[:N, 0], value.reshape(())


# speedup vs baseline: 1.2759x; 1.2759x over previous
import functools

import jax
import jax.numpy as jnp
from jax import lax
from jax.experimental import pallas as pl
from jax.experimental.pallas import tpu as pltpu
from jax.experimental.pallas import tpu_sc as plsc

NC = 2
NS = 16
NW = NC * NS
G = 128
NBUF = 4


def _mesh():
    return plsc.VectorSubcoreMesh(
        core_axis_name="c", subcore_axis_name="s", num_cores=NC, num_subcores=NS
    )


_SC_PARAMS = pltpu.CompilerParams(use_tc_tiling_on_sc=False)


def _core_split(ci, si, GP0, GP1):
    gpc = jnp.where(ci == 0, GP0, GP1)
    base = jnp.where(ci == 0, si * GP0, NS * GP0 + si * GP1)
    return gpc, base


def _make_deg(H, NP, GP0, GP1):
    RPS = NP // NS
    GPM = max(GP0, GP1)

    @functools.partial(
        pl.kernel,
        mesh=_mesh(),
        compiler_params=_SC_PARAMS,
        out_type=jax.ShapeDtypeStruct((NC, NP, H), jnp.float32),
        scratch_types=[
            pltpu.VMEM((GPM, G), jnp.int32),
            pltpu.VMEM((G, H), jnp.float32),
            pltpu.VMEM((RPS, H), jnp.float32),
            pltpu.VMEM_SHARED((NP, H), jnp.float32),
            pltpu.SemaphoreType.DMA((NBUF,)),
        ],
    )
    def k(dst_hbm, out_hbm, dstb, ones, zbuf, acc, ssem):
        ci = lax.axis_index("c")
        si = lax.axis_index("s")
        gpc, base = _core_split(ci, si, GP0, GP1)

        def zfill(i, _):
            zbuf[i, :] = jnp.zeros((H,), jnp.float32)
            return 0

        lax.fori_loop(0, RPS, zfill, 0)

        def ofill(i, _):
            ones[i, :] = jnp.ones((H,), jnp.float32)
            return 0

        lax.fori_loop(0, G, ofill, 0)

        @pl.when(ci == 0)
        def _():
            pltpu.sync_copy(dst_hbm.at[pl.ds(base, GP0)],
                            dstb.at[pl.ds(0, GP0)])

        @pl.when(ci == 1)
        def _():
            pltpu.sync_copy(dst_hbm.at[pl.ds(base, GP1)],
                            dstb.at[pl.ds(0, GP1)])

        pltpu.sync_copy(zbuf, acc.at[pl.ds(si * RPS, RPS)])
        plsc.subcore_barrier()

        for b in range(NBUF):
            pltpu.async_copy(ones, acc.at[dstb.at[b]], ssem.at[b], add=True)

        def outer(t, _):
            t0 = t * NBUF
            for b in range(NBUF):
                g = t0 + b
                pltpu.make_async_copy(ones, acc.at[dstb.at[b]],
                                      ssem.at[b]).wait()
                pltpu.async_copy(ones, acc.at[dstb.at[g]], ssem.at[b],
                                 add=True)
            return 0

        lax.fori_loop(1, gpc // NBUF, outer, 0)
        for b in range(NBUF):
            pltpu.make_async_copy(ones, acc.at[dstb.at[b]], ssem.at[b]).wait()
        plsc.subcore_barrier()
        pltpu.sync_copy(acc.at[pl.ds(si * RPS, RPS)],
                        out_hbm.at[ci, pl.ds(si * RPS, RPS)])

    return k


def _make_rowsum(H, NP, GP0, GP1):
    RPS = NP // NS
    GPM = max(GP0, GP1)

    @functools.partial(
        pl.kernel,
        mesh=_mesh(),
        compiler_params=_SC_PARAMS,
        out_type=jax.ShapeDtypeStruct((NC, NP, H), jnp.float32),
        scratch_types=[
            pltpu.VMEM((GPM, G), jnp.int32),
            pltpu.VMEM((GPM, G), jnp.int32),
            pltpu.VMEM((NBUF, G, H), jnp.float32),
            pltpu.VMEM((RPS, H), jnp.float32),
            pltpu.VMEM_SHARED((NP, H), jnp.float32),
            pltpu.SemaphoreType.DMA((NBUF,)),
            pltpu.SemaphoreType.DMA((NBUF,)),
        ],
    )
    def k(y_hbm, src_hbm, dst_hbm, out_hbm, srcb, dstb, rows, zbuf, acc,
          gsem, ssem):
        ci = lax.axis_index("c")
        si = lax.axis_index("s")
        gpc, base = _core_split(ci, si, GP0, GP1)

        def zfill(i, _):
            zbuf[i, :] = jnp.zeros((H,), jnp.float32)
            return 0

        lax.fori_loop(0, RPS, zfill, 0)

        def gather_work():
            for b in range(NBUF):
                pltpu.async_copy(y_hbm.at[srcb.at[b]], rows.at[b], gsem.at[b])
            pltpu.sync_copy(zbuf, acc.at[pl.ds(si * RPS, RPS)])
            plsc.subcore_barrier()

            def outer(t, _):
                t0 = t * NBUF
                for b in range(NBUF):
                    g = t0 + b
                    pltpu.make_async_copy(
                        y_hbm.at[srcb.at[g]], rows.at[b], gsem.at[b]).wait()
                    pltpu.async_copy(rows.at[b], acc.at[dstb.at[g]],
                                     ssem.at[b], add=True)
                for b in range(NBUF):
                    ng = t0 + NBUF + b

                    @pl.when(ng < gpc)
                    def _():
                        pltpu.make_async_copy(
                            rows.at[b], acc.at[dstb.at[b]], ssem.at[b]).wait()
                        pltpu.async_copy(y_hbm.at[srcb.at[ng]], rows.at[b],
                                         gsem.at[b])

                return 0

            lax.fori_loop(0, gpc // NBUF, outer, 0)
            for b in range(NBUF):
                pltpu.make_async_copy(rows.at[b], acc.at[dstb.at[b]],
                                      ssem.at[b]).wait()

        if GP1 == 0:
            @pl.when(ci == 0)
            def _():
                pltpu.sync_copy(src_hbm.at[pl.ds(base, GP0)],
                                srcb.at[pl.ds(0, GP0)])
                pltpu.sync_copy(dst_hbm.at[pl.ds(base, GP0)],
                                dstb.at[pl.ds(0, GP0)])
                gather_work()

            @pl.when(ci == 1)
            def _():
                pltpu.sync_copy(zbuf, acc.at[pl.ds(si * RPS, RPS)])
                plsc.subcore_barrier()
        else:
            @pl.when(ci == 0)
            def _():
                pltpu.sync_copy(src_hbm.at[pl.ds(base, GP0)],
                                srcb.at[pl.ds(0, GP0)])
                pltpu.sync_copy(dst_hbm.at[pl.ds(base, GP0)],
                                dstb.at[pl.ds(0, GP0)])

            @pl.when(ci == 1)
            def _():
                pltpu.sync_copy(src_hbm.at[pl.ds(base, GP1)],
                                srcb.at[pl.ds(0, GP1)])
                pltpu.sync_copy(dst_hbm.at[pl.ds(base, GP1)],
                                dstb.at[pl.ds(0, GP1)])

            gather_work()
        plsc.subcore_barrier()
        pltpu.sync_copy(acc.at[pl.ds(si * RPS, RPS)],
                        out_hbm.at[ci, pl.ds(si * RPS, RPS)])

    return k


def _k_xw(x, W1, N, NP, R):
    D = x.shape[1]
    H = W1.shape[1]
    grid = N // R

    def body(x_ref, w_ref, xw_ref):
        xw_ref[...] = jnp.dot(x_ref[...], w_ref[...],
                              preferred_element_type=jnp.float32)

    return pl.pallas_call(
        body,
        grid=(grid,),
        in_specs=[
            pl.BlockSpec((R, D), lambda i: (i, 0)),
            pl.BlockSpec((D, H), lambda i: (0, 0)),
        ],
        out_specs=pl.BlockSpec((R, H), lambda i: (i, 0)),
        out_shape=jax.ShapeDtypeStruct((NP, H), jnp.float32),
    )(x, W1)


def _k_scale(degbp, xwp, RP):
    NR = xwp.shape[0]
    grid = NR // RP

    def body(deg_ref, xw_ref, dinv_ref, y_ref):
        deg = deg_ref[0] + deg_ref[1] + 1.0
        dinvb = lax.rsqrt(deg)
        dinv_ref[...] = dinvb
        y_ref[...] = xw_ref[...] * dinvb

    return pl.pallas_call(
        body,
        grid=(grid,),
        in_specs=[
            pl.BlockSpec((NC, RP, 128), lambda i: (0, i, 0)),
            pl.BlockSpec((RP, 128), lambda i: (i, 0)),
        ],
        out_specs=[
            pl.BlockSpec((RP, 128), lambda i: (i, 0)),
            pl.BlockSpec((RP, 128), lambda i: (i, 0)),
        ],
        out_shape=[
            jax.ShapeDtypeStruct((NR, 128), jnp.float32),
            jax.ShapeDtypeStruct((NR, 128), jnp.float32),
        ],
    )(degbp, xwp)


def _k_layer(Sp, yp, dinvbp, bp, Wp, RP, with_hs, N):
    NR = yp.shape[0]
    grid = NR // RP

    def body(s_ref, y_ref, dinv_ref, b_ref, w_ref, o_ref, *hs_ref):
        S = s_ref[0] + s_ref[1]
        dinvb = dinv_ref[...]
        h = jnp.maximum(dinvb * (S + y_ref[...]) + b_ref[...], 0.0)
        o_ref[...] = jnp.dot(h, w_ref[...],
                             preferred_element_type=jnp.float32) * dinvb
        if with_hs:
            i = pl.program_id(0)
            r = lax.broadcasted_iota(jnp.int32, (RP, 128), 0) + i * RP
            nid = r * 8 + lax.broadcasted_iota(jnp.int32, (RP, 128), 1) // 16
            hm = jnp.where(nid < N, h, 0.0)
            hs_ref[0][...] = jnp.sum(hm, axis=0, keepdims=True)[None]

    out_specs = [pl.BlockSpec((RP, 128), lambda i: (i, 0))]
    out_shape = [jax.ShapeDtypeStruct((NR, 128), jnp.float32)]
    if with_hs:
        out_specs.append(pl.BlockSpec((1, 1, 128), lambda i: (i, 0, 0)))
        out_shape.append(jax.ShapeDtypeStruct((grid, 1, 128), jnp.float32))

    res = pl.pallas_call(
        body,
        grid=(grid,),
        in_specs=[
            pl.BlockSpec((NC, RP, 128), lambda i: (0, i, 0)),
            pl.BlockSpec((RP, 128), lambda i: (i, 0)),
            pl.BlockSpec((RP, 128), lambda i: (i, 0)),
            pl.BlockSpec((1, 128), lambda i: (0, 0)),
            pl.BlockSpec((128, 128), lambda i: (0, 0)),
        ],
        out_specs=out_specs,
        out_shape=out_shape,
    )(Sp, yp, dinvbp, bp, Wp)
    return res if with_hs else (res[0], None)


def _stage4(S3p, y3bp, dinvbp, b3r, hs, Wvp, bvr, N):
    NR = y3bp.shape[0]
    KB = hs.shape[0]

    def body(s_ref, y_ref, dinv_ref, b_ref, hs_ref, wv_ref, bv_ref,
             choice_ref, value_ref):
        S = s_ref[0] + s_ref[1]
        c = dinv_ref[...] * (S + y_ref[...]) + b_ref[0, 0]
        r = lax.broadcasted_iota(jnp.int32, (NR, 128), 0)
        nid = r * 8 + lax.broadcasted_iota(jnp.int32, (NR, 128), 1) // 16
        valid = nid < N
        c = jnp.where(valid, c, -jnp.inf)
        m = jnp.max(c)
        e = jnp.where(valid, jnp.exp(c - m), 0.0)
        choice_ref[...] = e * (16.0 / jnp.sum(e))
        hsum = jnp.sum(hs_ref[...], axis=0)
        value_ref[...] = (
            jnp.dot(hsum, wv_ref[...], preferred_element_type=jnp.float32)
            * (1.0 / N) + bv_ref[...]
        )

    return pl.pallas_call(
        body,
        in_specs=[
            pl.BlockSpec((NC, NR, 128), lambda: (0, 0, 0)),
            pl.BlockSpec((NR, 128), lambda: (0, 0)),
            pl.BlockSpec((NR, 128), lambda: (0, 0)),
            pl.BlockSpec((1, 1), lambda: (0, 0)),
            pl.BlockSpec((KB, 1, 128), lambda: (0, 0, 0)),
            pl.BlockSpec((128, 1), lambda: (0, 0)),
            pl.BlockSpec((1, 1), lambda: (0, 0)),
        ],
        out_specs=[
            pl.BlockSpec((NR, 128), lambda: (0, 0)),
            pl.BlockSpec((1, 1), lambda: (0, 0)),
        ],
        out_shape=[
            jax.ShapeDtypeStruct((NR, 128), jnp.float32),
            jax.ShapeDtypeStruct((1, 1), jnp.float32),
        ],
    )(S3p, y3bp, dinvbp, b3r, hs, Wvp, bvr)


def kernel(x, edge_index, W1, b1, W2, b2, W3, b3, Wv, bv):
    N, D = x.shape
    E = edge_index.shape[1]
    H = W1.shape[1]

    NP = ((N + 1 + 2047) // 2048) * 2048
    NR = NP * H // 128
    RP = NR // 5
    R = 2000
    TGS = ((-(-E // (NW * G)) + 7) // 8) * 8
    GP0 = ((TGS * 17 // 10) // 8) * 8
    GP1 = 2 * TGS - GP0
    DGP0 = ((TGS * 7 // 5) // 8) * 8
    DGP1 = 2 * TGS - DGP0
    EP = NS * 2 * TGS * G

    src = edge_index[0]
    dst = edge_index[1]
    pad = EP - E
    srcp = jnp.concatenate([src, jnp.zeros((pad,), jnp.int32)]).reshape(-1, G)
    dstp = jnp.concatenate([dst, jnp.full((pad,), N, jnp.int32)]).reshape(-1, G)

    eye8 = jnp.eye(8, dtype=jnp.float32)
    W2p = jnp.kron(eye8, W2)
    W3bp = jnp.kron(eye8, jnp.tile(W3, (1, H)))
    b1p = jnp.tile(b1, 8).reshape(1, 128)
    b2p = jnp.tile(b2, 8).reshape(1, 128)
    Wvp = jnp.tile(Wv, (8, 1))

    degb = _make_deg(H, NP, DGP0, DGP1)(dstp)
    xw = _k_xw(x, W1, N, NP, R)
    dinvbp, y1p = _k_scale(degb.reshape(NC, NR, 128), xw.reshape(NR, 128), RP)
    rowsum = _make_rowsum(H, NP, GP0, GP1)
    S1p = rowsum(y1p.reshape(NP, H), srcp, dstp)
    y2p, _ = _k_layer(S1p.reshape(NC, NR, 128), y1p, dinvbp, b1p, W2p,
                      RP, False, N)
    S2p = rowsum(y2p.reshape(NP, H), srcp, dstp)
    y3bp, hs = _k_layer(S2p.reshape(NC, NR, 128), y2p, dinvbp, b2p, W3bp,
                        RP, True, N)
    S3p = rowsum(y3bp.reshape(NP, H), srcp, dstp)
    choicebp, value = _stage4(
        S3p.reshape(NC, NR, 128), y3bp, dinvbp, b3.reshape(1, 1), hs,
        Wvp, bv.reshape(1, 1), N,
    )
    return choicebp.reshape(NP, H)[:N, 0], value.reshape(())
